# Initial kernel scaffold; baseline (speedup 1.0000x reference)
import jax
import jax.numpy as jnp
from jax.experimental import pallas as pl


def kernel(x1, edge_index1, batch1, cell, x1_sm, lens1, x2, edge_index2, batch2, x2_sm, lens2, params):
    raise NotImplementedError("write your pallas kernel here")


# speedup vs baseline: 8.5475x; 8.5475x over previous
import functools

import jax
import jax.numpy as jnp
from jax import lax
from jax.experimental import pallas as pl
from jax.experimental.pallas import tpu as pltpu
from jax.experimental.pallas import tpu_sc as plsc

_NS = 16
_ROWT = 640
_NPAD = _NS * _ROWT
_CH = 80


def _sc_deg(dst_flat, E):
    ept = E // _NS
    nch = ept // _CH
    mesh = plsc.VectorSubcoreMesh(core_axis_name="c", subcore_axis_name="s")

    def body(dst_hbm, out_hbm, didx_v, ones_v, stage_v, acc_sh):
        c = lax.axis_index("c")
        s = lax.axis_index("s")

        def fill_ones(i, carry):
            ones_v[i, :] = jnp.full((16,), 1.0, jnp.float32)
            return carry

        lax.fori_loop(0, _CH, fill_ones, 0)

        def fill_zero(i, carry):
            stage_v[i, :] = jnp.zeros((16,), jnp.float32)
            return carry

        lax.fori_loop(0, _CH, fill_zero, 0)

        def zero_acc(j, carry):
            pltpu.sync_copy(stage_v, acc_sh.at[pl.ds(s * _ROWT + j * _CH, _CH)])
            return carry

        lax.fori_loop(0, _ROWT // _CH, zero_acc, 0)
        plsc.subcore_barrier()

        def step(i, carry):
            base = c * E + s * ept + i * _CH
            pltpu.sync_copy(dst_hbm.at[pl.ds(base, _CH)], didx_v)
            pltpu.sync_copy(ones_v, acc_sh.at[didx_v], add=True)
            return carry

        lax.fori_loop(0, nch, step, 0)
        plsc.subcore_barrier()

        def copy_out(j, carry):
            off = s * _ROWT + j * _CH
            pltpu.sync_copy(acc_sh.at[pl.ds(off, _CH)], stage_v)
            pltpu.sync_copy(stage_v, out_hbm.at[c, pl.ds(off, _CH)])
            return carry

        lax.fori_loop(0, _ROWT // _CH, copy_out, 0)

    return pl.kernel(
        body,
        out_type=jax.ShapeDtypeStruct((2, _NPAD, 16), jnp.float32),
        mesh=mesh,
        scratch_types=[
            pltpu.VMEM((_CH,), jnp.int32),
            pltpu.VMEM((_CH, 16), jnp.float32),
            pltpu.VMEM((_CH, 16), jnp.float32),
            pltpu.VMEM_SHARED((_NPAD, 16), jnp.float32),
        ],
        name="sc_deg",
    )(dst_flat)


def _sc_agg(y_flat, src_flat, dst_flat, E, F):
    ept = E // _NS
    nch = ept // _CH
    mesh = plsc.VectorSubcoreMesh(core_axis_name="c", subcore_axis_name="s")

    def body(y_hbm, src_hbm, dst_hbm, out_hbm, sidx_v, didx_v, rows_v, sem, acc_sh):
        c = lax.axis_index("c")
        s = lax.axis_index("s")

        def zero_rows(i, carry):
            for j in range(F // 16):
                rows_v[i, pl.ds(j * 16, 16)] = jnp.zeros((16,), jnp.float32)
            return carry

        lax.fori_loop(0, _CH, zero_rows, 0)

        def zero_acc(j, carry):
            pltpu.sync_copy(rows_v, acc_sh.at[pl.ds(s * _ROWT + j * _CH, _CH)])
            return carry

        lax.fori_loop(0, _ROWT // _CH, zero_acc, 0)
        plsc.subcore_barrier()

        def step(i, carry):
            base = c * E + s * ept + i * _CH
            pltpu.sync_copy(src_hbm.at[pl.ds(base, _CH)], sidx_v)
            pltpu.sync_copy(dst_hbm.at[pl.ds(base, _CH)], didx_v)
            pltpu.async_copy(y_hbm.at[sidx_v], rows_v, sem).wait()
            pltpu.sync_copy(rows_v, acc_sh.at[didx_v], add=True)
            return carry

        lax.fori_loop(0, nch, step, 0)
        plsc.subcore_barrier()

        def copy_out(j, carry):
            off = s * _ROWT + j * _CH
            pltpu.sync_copy(acc_sh.at[pl.ds(off, _CH)], rows_v)
            pltpu.sync_copy(rows_v, out_hbm.at[c, pl.ds(off, _CH)])
            return carry

        lax.fori_loop(0, _ROWT // _CH, copy_out, 0)

    return pl.kernel(
        body,
        out_type=jax.ShapeDtypeStruct((2, _NPAD, F), jnp.float32),
        mesh=mesh,
        scratch_types=[
            pltpu.VMEM((_CH,), jnp.int32),
            pltpu.VMEM((_CH,), jnp.int32),
            pltpu.VMEM((_CH, F), jnp.float32),
            pltpu.SemaphoreType.DMA,
            pltpu.VMEM_SHARED((_NPAD, F), jnp.float32),
        ],
        name=f"sc_agg{F}",
        compiler_params=pltpu.CompilerParams(use_tc_tiling_on_sc=False),
    )(y_flat, src_flat, dst_flat)


_BR = 400


def _tc_prep(xs, deg):
    N = xs.shape[1]
    nb = N // _BR

    def body(x_ref, deg_ref, y_ref, d_ref):
        dv = lax.rsqrt(deg_ref[0][:, 0:1] + 1.0)
        d_ref[0] = dv
        y_ref[0] = x_ref[0] * dv

    return pl.pallas_call(
        body,
        grid=(2, nb),
        in_specs=[
            pl.BlockSpec((1, _BR, 80), lambda c, i: (c, i, 0)),
            pl.BlockSpec((1, _BR, 16), lambda c, i: (c, i, 0)),
        ],
        out_specs=[
            pl.BlockSpec((1, _BR, 80), lambda c, i: (c, i, 0)),
            pl.BlockSpec((1, _BR, 1), lambda c, i: (c, i, 0)),
        ],
        out_shape=[
            jax.ShapeDtypeStruct((2, N, 80), jnp.float32),
            jax.ShapeDtypeStruct((2, N, 1), jnp.float32),
        ],
    )(xs, deg)


def _tc_layer(agg, y, dinv, W, b, scale_out):
    N = y.shape[1]
    Fin = y.shape[2]
    Fout = W.shape[2]
    nb = N // _BR

    def body(agg_ref, y_ref, d_ref, w_ref, b_ref, o_ref):
        dv = d_ref[0]
        z = (agg_ref[0] + y_ref[0]) * dv
        h = jnp.dot(z, w_ref[0], preferred_element_type=jnp.float32) + b_ref[0]
        h = jnp.maximum(h, 0.0)
        if scale_out:
            h = h * dv
        o_ref[0] = h

    return pl.pallas_call(
        body,
        grid=(2, nb),
        in_specs=[
            pl.BlockSpec((1, _BR, Fin), lambda c, i: (c, i, 0)),
            pl.BlockSpec((1, _BR, Fin), lambda c, i: (c, i, 0)),
            pl.BlockSpec((1, _BR, 1), lambda c, i: (c, i, 0)),
            pl.BlockSpec((1, Fin, Fout), lambda c, i: (c, 0, 0)),
            pl.BlockSpec((1, 1, Fout), lambda c, i: (c, 0, 0)),
        ],
        out_specs=pl.BlockSpec((1, _BR, Fout), lambda c, i: (c, i, 0)),
        out_shape=jax.ShapeDtypeStruct((2, N, Fout), jnp.float32),
    )(agg, y, dinv, W, b)


_PCH = 512


def _tc_pool(h3, bounds):
    N = h3.shape[1]

    def body(bounds_ref, h_ref, o_ref, buf, sem):
        c = pl.program_id(0)
        seg = pl.program_id(1)
        start = bounds_ref[c, seg]
        end = bounds_ref[c, seg + 1]
        start_a = (start // 8) * 8
        nch = lax.div(end - start_a + (_PCH - 1), _PCH)

        def step(k, acc):
            off = jnp.minimum(start_a + k * _PCH, N - _PCH)
            cp = pltpu.make_async_copy(h_ref.at[c, pl.ds(off, _PCH)], buf, sem)
            cp.start()
            cp.wait()
            ids = off + lax.broadcasted_iota(jnp.int32, (_PCH, 1), 0)
            m = (ids >= start) & (ids < end)
            return jnp.maximum(acc, jnp.where(m, buf[...], -jnp.inf))

        acc = lax.fori_loop(
            0, nch, step, jnp.full((_PCH, 320), -jnp.inf, jnp.float32))
        o_ref[0, 0, 0] = jnp.max(acc, axis=0)

    return pl.pallas_call(
        body,
        grid=(2, 32),
        in_specs=[
            pl.BlockSpec(memory_space=pltpu.SMEM),
            pl.BlockSpec(memory_space=pltpu.HBM),
        ],
        out_specs=pl.BlockSpec((1, 1, 1, 320), lambda c, b: (c, b, 0, 0)),
        out_shape=jax.ShapeDtypeStruct((2, 32, 1, 320), jnp.float32),
        scratch_shapes=[
            pltpu.VMEM((_PCH, 320), jnp.float32),
            pltpu.SemaphoreType.DMA,
        ],
    )(bounds, h3)


_SQ5 = 0.7071067811865476


def _tx_flat(p):
    out = [p['fc']['w'], p['fc']['b'].reshape(1, -1)]
    for enc in (p['enc_prot'], p['enc_smi']):
        out += [enc['fc']['w'], enc['fc']['b'].reshape(1, -1)]
        for cp in enc['convs']:
            out += [cp['w'].transpose(2, 1, 0).reshape(9 * 128, 256),
                    cp['b'].reshape(1, -1)]
        out += [enc['ln']['g'].reshape(1, -1), enc['ln']['b'].reshape(1, -1)]
    for dec in (p['dec_smi'], p['dec_prot']):
        out += [dec['ft']['w'], dec['ft']['b'].reshape(1, -1)]
        for lp in dec['layers']:
            out += [lp['ln']['g'].reshape(1, -1), lp['ln']['b'].reshape(1, -1)]
            for ap in (lp['sa'], lp['ea']):
                for nm in ('wq', 'wk', 'wv', 'fc'):
                    out += [ap[nm]['w'], ap[nm]['b'].reshape(1, -1)]
            out += [lp['pf']['fc1']['w'], lp['pf']['fc1']['b'].reshape(1, -1),
                    lp['pf']['fc2']['w'], lp['pf']['fc2']['b'].reshape(1, -1)]
    return out


def _ln(x, g, b):
    m = jnp.mean(x, axis=-1, keepdims=True)
    v = jnp.mean((x - m) ** 2, axis=-1, keepdims=True)
    return (x - m) / jnp.sqrt(v + 1e-5) * g + b


def _dot(a, b):
    return jnp.dot(a, b, preferred_element_type=jnp.float32)


def _tx_body(*refs):
    o1_ref, o2_ref = refs[-2], refs[-1]
    x1s = refs[0][...]
    x2s = refs[1][...]
    it = iter(refs[2:-2])

    def nx():
        return next(it)[...]

    fw, fb = nx(), nx()
    d1 = _dot(x1s, fw) + fb
    d2 = _dot(x2s, fw) + fb

    def run_enc(d):
        ew, eb = nx(), nx()
        ci = _dot(d, ew) + eb
        for _ in range(3):
            cw, cb = nx(), nx()
            zp = jnp.zeros((4, 128), jnp.float32)
            xp = jnp.concatenate([zp, ci, zp], axis=0)
            xcol = jnp.concatenate([xp[k:k + 32] for k in range(9)], axis=1)
            conved = _dot(xcol, cw) + cb
            a = conved[:, :128]
            g = conved[:, 128:]
            ci = (a * jax.nn.sigmoid(g) + ci) * _SQ5
        lg, lb = nx(), nx()
        return _ln(ci, lg, lb)

    def attn(q, k, v, ps):
        (wq, bq), (wk, bk), (wv, bv), (wf, bf) = ps
        Q = _dot(q, wq) + bq
        K = _dot(k, wk) + bk
        V = _dot(v, wv) + bv
        outs = []
        for h in range(8):
            sl = slice(h * 16, (h + 1) * 16)
            e = lax.dot_general(Q[:, sl], K[:, sl], (((1,), (1,)), ((), ())),
                                preferred_element_type=jnp.float32) * 0.25
            a = jax.nn.softmax(e, axis=-1)
            outs.append(_dot(a, V[:, sl]))
        return _dot(jnp.concatenate(outs, axis=1), wf) + bf

    def run_dec(trg, src):
        tw, tb = nx(), nx()
        trg = _dot(trg, tw) + tb
        for _ in range(3):
            lg, lb = nx(), nx()
            sa = [(nx(), nx()) for _ in range(4)]
            ea = [(nx(), nx()) for _ in range(4)]
            f1w, f1b, f2w, f2b = nx(), nx(), nx(), nx()
            trg = _ln(trg + attn(trg, trg, trg, sa), lg, lb)
            trg = _ln(trg + attn(trg, src, src, ea), lg, lb)
            pf = _dot(jnp.maximum(_dot(trg, f1w) + f1b, 0.0), f2w) + f2b
            trg = _ln(trg + pf, lg, lb)
        return trg

    e1 = run_enc(d1)
    e2 = run_enc(d2)
    o2_ref[...] = run_dec(d2, e1)
    o1_ref[...] = run_dec(d1, e2)


def _tc_tx(x1_sm, x2_sm, params):
    flat = _tx_flat(params)
    outs = pl.pallas_call(
        _tx_body,
        out_shape=[
            jax.ShapeDtypeStruct((32, 128), jnp.float32),
            jax.ShapeDtypeStruct((32, 128), jnp.float32),
        ],
    )(x1_sm, x2_sm, *flat)
    return outs[0], outs[1]


def _tail_body(pool_ref, i1_ref, i2_ref, cell_ref,
               gw1, gb1, gw2, gb2, mw, mb,
               rw1, rb1, rw2, rb2, rw3, rb3,
               fw1, fb1, fw2, fb2, ow, ob, o_ref):
    h1 = jnp.maximum(_dot(pool_ref[0], gw1[0]) + gb1[0], 0.0)
    h1 = _dot(h1, gw2[0]) + gb2[0]
    h2 = jnp.maximum(_dot(pool_ref[1], gw1[1]) + gb1[1], 0.0)
    h2 = _dot(h2, gw2[1]) + gb2[1]
    h1 = h1 + _dot(i1_ref[...], mw[0]) + mb[0]
    h2 = h2 + _dot(i2_ref[...], mw[1]) + mb[1]
    cv = jnp.maximum(_dot(cell_ref[...], rw1[...]) + rb1[...], 0.0)
    cv = jnp.maximum(_dot(cv, rw2[...]) + rb2[...], 0.0)
    cv = _dot(cv, rw3[...]) + rb3[...]
    xc = h1 * h2 * cv
    xc = jnp.maximum(_dot(xc, fw1[...]) + fb1[...], 0.0)
    xc = jnp.maximum(_dot(xc, fw2[...]) + fb2[...], 0.0)
    o_ref[...] = _dot(xc, ow[...]) + ob[...]


def _tc_tail(pooled, inter1, inter2, cellp, tw):
    return pl.pallas_call(
        _tail_body,
        out_shape=jax.ShapeDtypeStruct((32, 8), jnp.float32),
    )(pooled, inter1, inter2, cellp, *tw)


def _pad2(w, r, c):
    return jnp.pad(w, ((0, r - w.shape[0]), (0, c - w.shape[1])))


def _padb(b, n):
    return jnp.pad(b, (0, n - b.shape[0])).reshape(1, n)


def kernel(x1, edge_index1, batch1, cell, x1_sm, lens1, x2, edge_index2,
           batch2, x2_sm, lens2, params):
    p = params
    N = x1.shape[0]

    E = edge_index1.shape[1]
    dst2 = jnp.concatenate([edge_index1[1], edge_index2[1]])
    src2 = jnp.concatenate([edge_index1[0], edge_index2[0] + N])

    deg = _sc_deg(dst2, E)

    xs = jnp.stack([jnp.pad(x1, ((0, 0), (0, 2))),
                    jnp.pad(x2, ((0, 0), (0, 2)))])
    y0, dinv = _tc_prep(xs, deg)

    W1 = jnp.stack([_pad2(p['g1c1']['w'], 80, 80), _pad2(p['g2c1']['w'], 80, 80)])
    B1 = jnp.stack([_padb(p['g1c1']['b'], 80), _padb(p['g2c1']['b'], 80)])
    W2 = jnp.stack([_pad2(p['g1c2']['w'], 80, 160), _pad2(p['g2c2']['w'], 80, 160)])
    B2 = jnp.stack([_padb(p['g1c2']['b'], 160), _padb(p['g2c2']['b'], 160)])
    W3 = jnp.stack([_pad2(p['g1c3']['w'], 160, 320), _pad2(p['g2c3']['w'], 160, 320)])
    B3 = jnp.stack([_padb(p['g1c3']['b'], 320), _padb(p['g2c3']['b'], 320)])

    agg = _sc_agg(y0.reshape(2 * N, 80), src2, dst2, E, 80)
    y1 = _tc_layer(agg, y0, dinv, W1, B1, True)
    agg = _sc_agg(y1.reshape(2 * N, 80), src2, dst2, E, 80)
    y2 = _tc_layer(agg, y1, dinv, W2, B2, True)
    agg = _sc_agg(y2.reshape(2 * N, 160), src2, dst2, E, 160)
    h3 = _tc_layer(agg, y2, dinv, W3, B3, False)

    qs = jnp.arange(33, dtype=jnp.int32)
    bounds = jnp.stack([
        jnp.searchsorted(batch1, qs).astype(jnp.int32),
        jnp.searchsorted(batch2, qs).astype(jnp.int32),
    ])
    pooled = _tc_pool(h3, bounds).reshape(2, 32, 320)

    inter1, inter2 = _tc_tx(x1_sm, x2_sm, p)

    tw = [
        jnp.stack([_pad2(p['g1f1']['w'], 320, 160), _pad2(p['g2f1']['w'], 320, 160)]),
        jnp.stack([_padb(p['g1f1']['b'], 160), _padb(p['g2f1']['b'], 160)]),
        jnp.stack([_pad2(p['g1f2']['w'], 160, 128), _pad2(p['g2f2']['w'], 160, 128)]),
        jnp.stack([_padb(p['g1f2']['b'], 128), _padb(p['g2f2']['b'], 128)]),
        jnp.stack([p['map1']['w'], p['map2']['w']]),
        jnp.stack([p['map1']['b'].reshape(1, 128), p['map2']['b'].reshape(1, 128)]),
        _pad2(p['red1']['w'], 960, 512), p['red1']['b'].reshape(1, 512),
        p['red2']['w'], p['red2']['b'].reshape(1, 256),
        p['red3']['w'], p['red3']['b'].reshape(1, 128),
        p['fc1']['w'], p['fc1']['b'].reshape(1, 512),
        p['fc2']['w'], p['fc2']['b'].reshape(1, 128),
        _pad2(p['out']['w'], 128, 8), _padb(p['out']['b'], 8),
    ]
    cellp = jnp.pad(cell, ((0, 0), (0, 6)))
    out8 = _tc_tail(pooled, inter1, inter2, cellp, tw)
    return out8[:, :2]
